# speedup vs baseline: 25.8802x; 25.8802x over previous
import jax
import jax.numpy as jnp
from jax.experimental import pallas as pl
from jax.experimental.pallas import tpu as pltpu

_B, _T = 16384, 4096
_BLOCK_ROWS = 512


def _copy_swap_body(x_ref, o_ref):
    x = x_ref[...]
    o_ref[...] = x
    o_ref[:, 0:1] = x[:, 1:2]
    o_ref[:, 1:2] = x[:, 0:1]


def kernel(tokens):
    return pl.pallas_call(
        _copy_swap_body,
        grid=(_B // _BLOCK_ROWS,),
        in_specs=[pl.BlockSpec((_BLOCK_ROWS, _T), lambda i: (i, 0))],
        out_specs=pl.BlockSpec((_BLOCK_ROWS, _T), lambda i: (i, 0)),
        out_shape=jax.ShapeDtypeStruct((_B, _T), tokens.dtype),
    )(tokens)
